# baseline (device time: 83969 ns/iter reference)
import jax
import jax.numpy as jnp
from jax import lax
from jax.experimental import pallas as pl
from jax.experimental.pallas import tpu as pltpu

N_DEV = 4
B = 2
SQ = 256
SKV_SH = 256
SKV = N_DEV * SKV_SH
H_LOC = 4
DH = 64
D_MODEL = 512


def kernel(x, Wq, K_ext, V_ext, Wo):
    x2d = x.reshape(B * SQ, D_MODEL)
    K_t = K_ext.transpose(0, 2, 1, 3)
    V_t = V_ext.transpose(0, 2, 1, 3)

    def body(x_ref, wq_ref, k_ref, v_ref, wo_ref, out_ref,
             kfull, vfull, pbuf, accbuf,
             send_k, recv_k, send_v, recv_v, send_p, recv_p):
        my = lax.axis_index("i")

        bsem = pltpu.get_barrier_semaphore()
        for k in range(1, N_DEV):
            pl.semaphore_signal(
                bsem, inc=1,
                device_id=((my + k) % N_DEV,),
                device_id_type=pl.DeviceIdType.MESH,
            )
        pl.semaphore_wait(bsem, N_DEV - 1)

        kv_rdmas = []
        for k in range(1, N_DEV):
            dest = (my + k) % N_DEV
            rk = pltpu.make_async_remote_copy(
                src_ref=k_ref.at[:, pl.ds(dest * H_LOC, H_LOC), :, :],
                dst_ref=kfull.at[:, :, pl.ds(my * SKV_SH, SKV_SH), :],
                send_sem=send_k.at[k - 1],
                recv_sem=recv_k.at[k - 1],
                device_id=(dest,),
                device_id_type=pl.DeviceIdType.MESH,
            )
            rk.start()
            rv = pltpu.make_async_remote_copy(
                src_ref=v_ref.at[:, pl.ds(dest * H_LOC, H_LOC), :, :],
                dst_ref=vfull.at[:, :, pl.ds(my * SKV_SH, SKV_SH), :],
                send_sem=send_v.at[k - 1],
                recv_sem=recv_v.at[k - 1],
                device_id=(dest,),
                device_id_type=pl.DeviceIdType.MESH,
            )
            rv.start()
            kv_rdmas.append((rk, rv))

        kfull[:, :, pl.ds(my * SKV_SH, SKV_SH), :] = (
            k_ref[:, pl.ds(my * H_LOC, H_LOC), :, :]
        )
        vfull[:, :, pl.ds(my * SKV_SH, SKV_SH), :] = (
            v_ref[:, pl.ds(my * H_LOC, H_LOC), :, :]
        )

        q = jnp.dot(x_ref[...], wq_ref[...],
                    preferred_element_type=jnp.float32)

        for rk, rv in kv_rdmas:
            rk.wait()
            rv.wait()

        qi = lax.broadcasted_iota(jnp.int32, (SQ, SKV), 0)
        ki = lax.broadcasted_iota(jnp.int32, (SQ, SKV), 1)
        mask = (jnp.abs(qi - ki) <= 128) | (ki < 32) | (qi < 32)

        ctx_rows = []
        for b in range(B):
            ctx_cols = []
            for h in range(H_LOC):
                qbh = q[b * SQ:(b + 1) * SQ, h * DH:(h + 1) * DH]
                kbh = kfull[b, h]
                s = lax.dot_general(
                    qbh, kbh, (((1,), (1,)), ((), ())),
                    preferred_element_type=jnp.float32,
                ) * 0.125
                s = jnp.where(mask, s, -1e9)
                m = jnp.max(s, axis=1, keepdims=True)
                w = jnp.exp(s - m)
                w = w / jnp.sum(w, axis=1, keepdims=True)
                ctx_cols.append(
                    jnp.dot(w, vfull[b, h],
                            preferred_element_type=jnp.float32)
                )
            ctx_rows.append(jnp.concatenate(ctx_cols, axis=1))
        ctx2d = jnp.concatenate(ctx_rows, axis=0)

        partial = jnp.dot(ctx2d, wo_ref[...],
                          preferred_element_type=jnp.float32)
        pbuf[...] = partial

        p_rdmas = []
        for k in range(1, N_DEV):
            dest = (my + k) % N_DEV
            r = pltpu.make_async_remote_copy(
                src_ref=pbuf,
                dst_ref=accbuf.at[k - 1],
                send_sem=send_p.at[k - 1],
                recv_sem=recv_p.at[k - 1],
                device_id=(dest,),
                device_id_type=pl.DeviceIdType.MESH,
            )
            r.start()
            p_rdmas.append(r)
        for r in p_rdmas:
            r.wait()

        out_ref[...] = pbuf[...] + accbuf[0] + accbuf[1] + accbuf[2]

    out2d = pl.pallas_call(
        body,
        out_shape=jax.ShapeDtypeStruct((B * SQ, D_MODEL), jnp.float32),
        in_specs=[pl.BlockSpec(memory_space=pltpu.VMEM)] * 5,
        out_specs=pl.BlockSpec(memory_space=pltpu.VMEM),
        scratch_shapes=[
            pltpu.VMEM((B, H_LOC, SKV, DH), jnp.float32),
            pltpu.VMEM((B, H_LOC, SKV, DH), jnp.float32),
            pltpu.VMEM((B * SQ, D_MODEL), jnp.float32),
            pltpu.VMEM((N_DEV - 1, B * SQ, D_MODEL), jnp.float32),
            pltpu.SemaphoreType.DMA((N_DEV - 1,)),
            pltpu.SemaphoreType.DMA((N_DEV - 1,)),
            pltpu.SemaphoreType.DMA((N_DEV - 1,)),
            pltpu.SemaphoreType.DMA((N_DEV - 1,)),
            pltpu.SemaphoreType.DMA((N_DEV - 1,)),
            pltpu.SemaphoreType.DMA((N_DEV - 1,)),
        ],
        compiler_params=pltpu.CompilerParams(collective_id=0),
    )(x2d, Wq, K_t, V_t, Wo)

    return out2d.reshape(B, SQ, D_MODEL)


# device time: 51189 ns/iter; 1.6404x vs baseline; 1.6404x over previous
import jax
import jax.numpy as jnp
from jax import lax
from jax.experimental import pallas as pl
from jax.experimental.pallas import tpu as pltpu

N_DEV = 4
B = 2
SQ = 256
SKV_SH = 256
SKV = N_DEV * SKV_SH
H_LOC = 4
DH = 64
D_MODEL = 512

SEND_ORDER = (2, 1, 3)
RECV_ORDER = (1, 3, 2)


def kernel(x, Wq, K_ext, V_ext, Wo):
    x2d = x.reshape(B * SQ, D_MODEL)
    K_c = K_ext.transpose(0, 2, 1, 3).astype(jnp.bfloat16)
    V_c = V_ext.transpose(0, 2, 1, 3).astype(jnp.bfloat16)

    def body(x_ref, wq_ref, k_ref, v_ref, wo_ref, out_ref,
             kbuf, vbuf, pbuf, accbuf,
             send_k, recv_k, send_v, recv_v, send_p, recv_p):
        my = lax.axis_index("i")

        bsem = pltpu.get_barrier_semaphore()
        for k in range(1, N_DEV):
            pl.semaphore_signal(
                bsem, inc=1,
                device_id=((my + k) % N_DEV,),
                device_id_type=pl.DeviceIdType.MESH,
            )
        pl.semaphore_wait(bsem, N_DEV - 1)

        rdmas = {}
        for k in SEND_ORDER:
            dest = (my + k) % N_DEV
            rk = pltpu.make_async_remote_copy(
                src_ref=k_ref.at[:, pl.ds(dest * H_LOC, H_LOC), :, :],
                dst_ref=kbuf.at[k - 1],
                send_sem=send_k.at[k - 1],
                recv_sem=recv_k.at[k - 1],
                device_id=(dest,),
                device_id_type=pl.DeviceIdType.MESH,
            )
            rk.start()
            rv = pltpu.make_async_remote_copy(
                src_ref=v_ref.at[:, pl.ds(dest * H_LOC, H_LOC), :, :],
                dst_ref=vbuf.at[k - 1],
                send_sem=send_v.at[k - 1],
                recv_sem=recv_v.at[k - 1],
                device_id=(dest,),
                device_id_type=pl.DeviceIdType.MESH,
            )
            rv.start()
            rdmas[k] = (rk, rv)

        q = jnp.dot(x_ref[...], wq_ref[...],
                    preferred_element_type=jnp.float32)
        qb = q.astype(jnp.bfloat16)

        qi = lax.broadcasted_iota(jnp.int32, (SQ, SKV_SH), 0)
        ki_loc = lax.broadcasted_iota(jnp.int32, (SQ, SKV_SH), 1)

        acc = [[None] * H_LOC for _ in range(B)]
        den = [[None] * H_LOC for _ in range(B)]

        def process_chunk(origin, get_k, get_v, first):
            ki = ki_loc + origin * SKV_SH
            mask = (jnp.abs(qi - ki) <= 128) | (ki < 32) | (qi < 32)
            for b in range(B):
                for h in range(H_LOC):
                    qbh = qb[b * SQ:(b + 1) * SQ, h * DH:(h + 1) * DH]
                    s = lax.dot_general(
                        qbh, get_k(b, h), (((1,), (1,)), ((), ())),
                        preferred_element_type=jnp.float32,
                    ) * 0.125
                    p = jnp.exp(jnp.where(mask, s, -1e9))
                    pv = jnp.dot(p.astype(jnp.bfloat16), get_v(b, h),
                                 preferred_element_type=jnp.float32)
                    rs = jnp.sum(p, axis=1, keepdims=True)
                    if first:
                        acc[b][h] = pv
                        den[b][h] = rs
                    else:
                        acc[b][h] = acc[b][h] + pv
                        den[b][h] = den[b][h] + rs

        process_chunk(
            my,
            lambda b, h: k_ref[b, my * H_LOC + h],
            lambda b, h: v_ref[b, my * H_LOC + h],
            first=True,
        )

        for k in RECV_ORDER:
            rk, rv = rdmas[k]
            rk.wait_recv()
            rv.wait_recv()
            origin = (my - k) % N_DEV
            process_chunk(
                origin,
                lambda b, h, _k=k: kbuf[_k - 1, b, h],
                lambda b, h, _k=k: vbuf[_k - 1, b, h],
                first=False,
            )

        ctx_rows = []
        for b in range(B):
            ctx_rows.append(jnp.concatenate(
                [acc[b][h] / den[b][h] for h in range(H_LOC)], axis=1))
        ctx2d = jnp.concatenate(ctx_rows, axis=0)

        partial = jnp.dot(ctx2d, wo_ref[...],
                          preferred_element_type=jnp.float32)
        pbuf[...] = partial.astype(jnp.bfloat16)

        p_rdmas = []
        for k in SEND_ORDER:
            dest = (my + k) % N_DEV
            r = pltpu.make_async_remote_copy(
                src_ref=pbuf,
                dst_ref=accbuf.at[k - 1],
                send_sem=send_p.at[k - 1],
                recv_sem=recv_p.at[k - 1],
                device_id=(dest,),
                device_id_type=pl.DeviceIdType.MESH,
            )
            r.start()
            p_rdmas.append(r)
        for r in p_rdmas:
            r.wait_recv()

        out_ref[...] = partial + (
            accbuf[0].astype(jnp.float32)
            + accbuf[1].astype(jnp.float32)
            + accbuf[2].astype(jnp.float32)
        )

        for k in SEND_ORDER:
            rk, rv = rdmas[k]
            rk.wait_send()
            rv.wait_send()
        for r in p_rdmas:
            r.wait_send()

    out2d = pl.pallas_call(
        body,
        out_shape=jax.ShapeDtypeStruct((B * SQ, D_MODEL), jnp.float32),
        in_specs=[pl.BlockSpec(memory_space=pltpu.VMEM)] * 5,
        out_specs=pl.BlockSpec(memory_space=pltpu.VMEM),
        scratch_shapes=[
            pltpu.VMEM((N_DEV - 1, B, H_LOC, SKV_SH, DH), jnp.bfloat16),
            pltpu.VMEM((N_DEV - 1, B, H_LOC, SKV_SH, DH), jnp.bfloat16),
            pltpu.VMEM((B * SQ, D_MODEL), jnp.bfloat16),
            pltpu.VMEM((N_DEV - 1, B * SQ, D_MODEL), jnp.bfloat16),
            pltpu.SemaphoreType.DMA((N_DEV - 1,)),
            pltpu.SemaphoreType.DMA((N_DEV - 1,)),
            pltpu.SemaphoreType.DMA((N_DEV - 1,)),
            pltpu.SemaphoreType.DMA((N_DEV - 1,)),
            pltpu.SemaphoreType.DMA((N_DEV - 1,)),
            pltpu.SemaphoreType.DMA((N_DEV - 1,)),
        ],
        compiler_params=pltpu.CompilerParams(collective_id=0),
    )(x2d, Wq, K_c, V_c, Wo)

    return out2d.reshape(B, SQ, D_MODEL)
